# Initial kernel scaffold; baseline (speedup 1.0000x reference)
import jax
import jax.numpy as jnp
from jax.experimental import pallas as pl


def kernel(coords, connectivity_tensor, y, du, dv):
    raise NotImplementedError("write your pallas kernel here")


# speedup vs baseline: 15.1931x; 15.1931x over previous
import functools

import jax
import jax.numpy as jnp
from jax import lax
from jax.experimental import pallas as pl
from jax.experimental.pallas import tpu as pltpu, tpu_sc as plsc

DIM = 3
N_NODES = 100000
MAX_NEIGHBORS = 32
LANES = 16
NUM_WORKERS = 32
CHUNK = 512
NUM_CHUNKS = -(-N_NODES // CHUNK)
ITERS = -(-NUM_CHUNKS // NUM_WORKERS)
EPS = 1e-8
LAM = 1e-6


def _body(dv_hbm, du_hbm, ox_hbm, oy_hbm, oz_hbm, dv_v, du_v, ox_v, oy_v, oz_v):
    wid = lax.axis_index("c") * 16 + lax.axis_index("s")
    iota = jnp.arange(LANES, dtype=jnp.int32)
    groups = CHUNK // LANES

    for it in range(ITERS):
        j = wid + it * NUM_WORKERS

        @pl.when(j * CHUNK < N_NODES)
        def _chunk():
            start = jnp.minimum(j * CHUNK, N_NODES - CHUNK)
            pltpu.sync_copy(dv_hbm.at[pl.ds(start * 96, CHUNK * 96)], dv_v)
            pltpu.sync_copy(du_hbm.at[pl.ds(start * 32, CHUNK * 32)], du_v)

            def group(g, _):
                node = g * LANES + iota
                dvb = node * 96
                dub = node * 32
                axx = ayy = azz = axy = axz = ayz = None
                bx = by = bz = None
                for n in range(MAX_NEIGHBORS):
                    ix = dvb + (3 * n)
                    x = plsc.load_gather(dv_v, [ix])
                    y = plsc.load_gather(dv_v, [ix + 1])
                    z = plsc.load_gather(dv_v, [ix + 2])
                    d = plsc.load_gather(du_v, [dub + n])
                    w = 1.0 / (x * x + y * y + z * z + EPS)
                    wx = w * x
                    wy = w * y
                    wz = w * z
                    if n == 0:
                        axx, ayy, azz = wx * x, wy * y, wz * z
                        axy, axz, ayz = wx * y, wx * z, wy * z
                        bx, by, bz = wx * d, wy * d, wz * d
                    else:
                        axx += wx * x
                        ayy += wy * y
                        azz += wz * z
                        axy += wx * y
                        axz += wx * z
                        ayz += wy * z
                        bx += wx * d
                        by += wy * d
                        bz += wz * d
                axx += LAM
                ayy += LAM
                azz += LAM
                c00 = ayy * azz - ayz * ayz
                c01 = axz * ayz - axy * azz
                c02 = axy * ayz - axz * ayy
                c11 = axx * azz - axz * axz
                c12 = axy * axz - axx * ayz
                c22 = axx * ayy - axy * axy
                inv_det = 1.0 / (axx * c00 + axy * c01 + axz * c02)
                sl = pl.ds(g * LANES, LANES)
                ox_v[sl] = (c00 * bx + c01 * by + c02 * bz) * inv_det
                oy_v[sl] = (c01 * bx + c11 * by + c12 * bz) * inv_det
                oz_v[sl] = (c02 * bx + c12 * by + c22 * bz) * inv_det
                return ()

            lax.fori_loop(0, groups, group, ())
            pltpu.sync_copy(ox_v, ox_hbm.at[pl.ds(start, CHUNK)])
            pltpu.sync_copy(oy_v, oy_hbm.at[pl.ds(start, CHUNK)])
            pltpu.sync_copy(oz_v, oz_hbm.at[pl.ds(start, CHUNK)])


@jax.jit
def _ls_grads(dv_flat, du_flat):
    f32 = jnp.float32
    run = pl.kernel(
        _body,
        out_type=(
            jax.ShapeDtypeStruct((N_NODES,), f32),
            jax.ShapeDtypeStruct((N_NODES,), f32),
            jax.ShapeDtypeStruct((N_NODES,), f32),
        ),
        mesh=plsc.VectorSubcoreMesh(core_axis_name="c", subcore_axis_name="s"),
        compiler_params=pltpu.CompilerParams(needs_layout_passes=False),
        scratch_types=[
            pltpu.VMEM((CHUNK * 96,), f32),
            pltpu.VMEM((CHUNK * 32,), f32),
            pltpu.VMEM((CHUNK,), f32),
            pltpu.VMEM((CHUNK,), f32),
            pltpu.VMEM((CHUNK,), f32),
        ],
    )
    return run(dv_flat, du_flat)


def kernel(coords, connectivity_tensor, y, du, dv):
    del coords, connectivity_tensor, y
    gx, gy, gz = _ls_grads(dv.reshape(-1), du.reshape(-1))
    return (gx[:, None], gy[:, None], gz[:, None])


# speedup vs baseline: 526.3123x; 34.6416x over previous
import jax
import jax.numpy as jnp
from jax import lax
from jax.experimental import pallas as pl
from jax.experimental.pallas import tpu as pltpu, tpu_sc as plsc

DIM = 3
N_NODES = 100000
NBR = 32
LANES = 16
NUM_WORKERS = 32
CHUNK = 256
FULL_CHUNKS = N_NODES // CHUNK
TAIL = N_NODES - FULL_CHUNKS * CHUNK
ITERS = -(-(FULL_CHUNKS + 1) // NUM_WORKERS)
EPS = 1e-8
LAM = 1e-6


def _accum_solve(loads, store, g):
    axx = ayy = azz = axy = axz = ayz = None
    bx = by = bz = None
    for n in range(NBR):
        x, y, z, d = loads(n)
        w = 1.0 / (x * x + y * y + z * z + EPS)
        wx = w * x
        wy = w * y
        wz = w * z
        if n == 0:
            axx, ayy, azz = wx * x, wy * y, wz * z
            axy, axz, ayz = wx * y, wx * z, wy * z
            bx, by, bz = wx * d, wy * d, wz * d
        else:
            axx += wx * x
            ayy += wy * y
            azz += wz * z
            axy += wx * y
            axz += wx * z
            ayz += wy * z
            bx += wx * d
            by += wy * d
            bz += wz * d
    axx += LAM
    ayy += LAM
    azz += LAM
    c00 = ayy * azz - ayz * ayz
    c01 = axz * ayz - axy * azz
    c02 = axy * ayz - axz * ayy
    c11 = axx * azz - axz * axz
    c12 = axy * axz - axx * ayz
    c22 = axx * ayy - axy * axy
    inv_det = 1.0 / (axx * c00 + axy * c01 + axz * c02)
    store((c00 * bx + c01 * by + c02 * bz) * inv_det,
          (c01 * bx + c11 * by + c12 * bz) * inv_det,
          (c02 * bx + c12 * by + c22 * bz) * inv_det)


def _body(dvt_hbm, dut_hbm, dvf_hbm, duf_hbm, ox_hbm, oy_hbm, oz_hbm,
          dv_v, du_v, ox_v, oy_v, oz_v,
          dv_t, du_t, ox_t, oy_t, oz_t, sem):
    wid = lax.axis_index("c") * 16 + lax.axis_index("s")

    def process_chunk(j):
        s = pl.multiple_of(j * CHUNK, 128)
        copies = []
        for i in range(DIM):
            for t in range(NBR // 8):
                copies.append(pltpu.async_copy(
                    dvt_hbm.at[i, pl.ds(t * 8, 8), pl.ds(s, CHUNK)],
                    dv_v.at[i, t], sem))
        for t in range(NBR // 8):
            copies.append(pltpu.async_copy(
                dut_hbm.at[pl.ds(t * 8, 8), pl.ds(s, CHUNK)],
                du_v.at[t], sem))
        for c in copies:
            c.wait()

        def group(g, _):
            sl = pl.ds(g * LANES, LANES)

            def loads(n):
                t, n8 = divmod(n, 8)
                return (dv_v[0, t, n8, sl], dv_v[1, t, n8, sl],
                        dv_v[2, t, n8, sl], du_v[t, n8, sl])

            def store(gx, gy, gz):
                ox_v[sl] = gx
                oy_v[sl] = gy
                oz_v[sl] = gz

            _accum_solve(loads, store, g)
            return ()

        lax.fori_loop(0, CHUNK // LANES, group, ())
        pltpu.sync_copy(ox_v, ox_hbm.at[pl.ds(s, CHUNK)])
        pltpu.sync_copy(oy_v, oy_hbm.at[pl.ds(s, CHUNK)])
        pltpu.sync_copy(oz_v, oz_hbm.at[pl.ds(s, CHUNK)])

    def process_tail():
        pltpu.sync_copy(dvf_hbm, dv_t)
        pltpu.sync_copy(duf_hbm, du_t)
        iota = jnp.arange(LANES, dtype=jnp.int32)

        def group(g, _):
            sl = pl.ds(g * LANES, LANES)
            node = g * LANES + iota
            dvb = node * (3 * NBR)
            dub = node * NBR

            def loads(n):
                ix = dvb + 3 * n
                return (plsc.load_gather(dv_t, [ix]),
                        plsc.load_gather(dv_t, [ix + 1]),
                        plsc.load_gather(dv_t, [ix + 2]),
                        plsc.load_gather(du_t, [dub + n]))

            def store(gx, gy, gz):
                ox_t[sl] = gx
                oy_t[sl] = gy
                oz_t[sl] = gz

            _accum_solve(loads, store, g)
            return ()

        lax.fori_loop(0, TAIL // LANES, group, ())
        base = FULL_CHUNKS * CHUNK
        pltpu.sync_copy(ox_t, ox_hbm.at[pl.ds(base, TAIL)])
        pltpu.sync_copy(oy_t, oy_hbm.at[pl.ds(base, TAIL)])
        pltpu.sync_copy(oz_t, oz_hbm.at[pl.ds(base, TAIL)])

    for it in range(ITERS):
        j = wid + it * NUM_WORKERS
        if (it + 1) * NUM_WORKERS <= FULL_CHUNKS:
            process_chunk(j)
        else:

            @pl.when(j < FULL_CHUNKS)
            def _full():
                process_chunk(j)

            @pl.when(j == FULL_CHUNKS)
            def _tail():
                process_tail()


@jax.jit
def _ls_grads(dvt, dut, dvf, duf):
    f32 = jnp.float32
    run = pl.kernel(
        _body,
        out_type=(
            jax.ShapeDtypeStruct((N_NODES,), f32),
            jax.ShapeDtypeStruct((N_NODES,), f32),
            jax.ShapeDtypeStruct((N_NODES,), f32),
        ),
        mesh=plsc.VectorSubcoreMesh(core_axis_name="c", subcore_axis_name="s"),
        compiler_params=pltpu.CompilerParams(needs_layout_passes=False),
        scratch_types=[
            pltpu.VMEM((DIM, NBR // 8, 8, CHUNK), f32),
            pltpu.VMEM((NBR // 8, 8, CHUNK), f32),
            pltpu.VMEM((CHUNK,), f32),
            pltpu.VMEM((CHUNK,), f32),
            pltpu.VMEM((CHUNK,), f32),
            pltpu.VMEM((TAIL * 3 * NBR,), f32),
            pltpu.VMEM((TAIL * NBR,), f32),
            pltpu.VMEM((TAIL,), f32),
            pltpu.VMEM((TAIL,), f32),
            pltpu.VMEM((TAIL,), f32),
            pltpu.SemaphoreType.DMA,
        ],
    )
    return run(dvt, dut, dvf, duf)


def kernel(coords, connectivity_tensor, y, du, dv):
    del coords, connectivity_tensor, y
    dvt = jnp.transpose(dv, (2, 1, 0))
    dut = jnp.transpose(du[:, :, 0], (1, 0))
    base = FULL_CHUNKS * CHUNK
    dvf = dv[base:].reshape(-1)
    duf = du[base:].reshape(-1)
    gx, gy, gz = _ls_grads(dvt, dut, dvf, duf)
    return (gx[:, None], gy[:, None], gz[:, None])


# speedup vs baseline: 645.2343x; 1.2260x over previous
import jax
import jax.numpy as jnp
from jax import lax
from jax.experimental import pallas as pl
from jax.experimental.pallas import tpu as pltpu, tpu_sc as plsc

DIM = 3
N_NODES = 100000
NBR = 32
LANES = 16
NUM_WORKERS = 32
CHUNK = 256
FULL_CHUNKS = N_NODES // CHUNK
TAIL = N_NODES - FULL_CHUNKS * CHUNK
EPS = 1e-8
LAM = 1e-6


def _accum_solve(loads, store, g):
    axx = ayy = azz = axy = axz = ayz = None
    bx = by = bz = None
    for n in range(NBR):
        x, y, z, d = loads(n)
        w = 1.0 / (x * x + y * y + z * z + EPS)
        wx = w * x
        wy = w * y
        wz = w * z
        if n == 0:
            axx, ayy, azz = wx * x, wy * y, wz * z
            axy, axz, ayz = wx * y, wx * z, wy * z
            bx, by, bz = wx * d, wy * d, wz * d
        else:
            axx += wx * x
            ayy += wy * y
            azz += wz * z
            axy += wx * y
            axz += wx * z
            ayz += wy * z
            bx += wx * d
            by += wy * d
            bz += wz * d
    axx += LAM
    ayy += LAM
    azz += LAM
    c00 = ayy * azz - ayz * ayz
    c01 = axz * ayz - axy * azz
    c02 = axy * ayz - axz * ayy
    c11 = axx * azz - axz * axz
    c12 = axy * axz - axx * ayz
    c22 = axx * ayy - axy * axy
    inv_det = 1.0 / (axx * c00 + axy * c01 + axz * c02)
    store((c00 * bx + c01 * by + c02 * bz) * inv_det,
          (c01 * bx + c11 * by + c12 * bz) * inv_det,
          (c02 * bx + c12 * by + c22 * bz) * inv_det)


def _body(dvt_hbm, dut_hbm, dvf_hbm, duf_hbm, ox_hbm, oy_hbm, oz_hbm,
          dv_v, du_v, ox_v, oy_v, oz_v,
          dv_t, du_t, ox_t, oy_t, oz_t,
          sem_a, sem_b, sem_oa, sem_ob):
    wid = lax.axis_index("c") * 16 + lax.axis_index("s")
    sems = (sem_a, sem_b)
    osems = (sem_oa, sem_ob)
    outs = ((ox_v, ox_hbm), (oy_v, oy_hbm), (oz_v, oz_hbm))

    def chunk_start(m):
        return pl.multiple_of((wid + m * NUM_WORKERS) * CHUNK, 128)

    def start_in(m, slot):
        s = chunk_start(m)
        for i in range(DIM):
            pltpu.async_copy(dvt_hbm.at[i, :, pl.ds(s, CHUNK)],
                             dv_v.at[slot, i], sems[slot])
        pltpu.async_copy(dut_hbm.at[:, pl.ds(s, CHUNK)], du_v.at[slot], sems[slot])

    def wait_in(slot):
        for i in range(DIM):
            pltpu.make_async_copy(dvt_hbm.at[i, :, pl.ds(0, CHUNK)],
                                  dv_v.at[slot, i], sems[slot]).wait()
        pltpu.make_async_copy(dut_hbm.at[:, pl.ds(0, CHUNK)],
                              du_v.at[slot], sems[slot]).wait()

    def start_out(m, slot):
        s = chunk_start(m)
        for buf, hbm in outs:
            pltpu.async_copy(buf.at[slot], hbm.at[pl.ds(s, CHUNK)], osems[slot])

    def wait_out(slot):
        for buf, hbm in outs:
            pltpu.make_async_copy(buf.at[slot], hbm.at[pl.ds(0, CHUNK)],
                                  osems[slot]).wait()

    def compute(slot):
        @plsc.parallel_loop(0, CHUNK // LANES, unroll=2)
        def group(g):
            sl = pl.ds(g * LANES, LANES)

            def loads(n):
                return (dv_v[slot, 0, n, sl], dv_v[slot, 1, n, sl],
                        dv_v[slot, 2, n, sl], du_v[slot, n, sl])

            def store(gx, gy, gz):
                ox_v[slot, sl] = gx
                oy_v[slot, sl] = gy
                oz_v[slot, sl] = gz

            _accum_solve(loads, store, g)

    def process_tail():
        pltpu.sync_copy(dvf_hbm, dv_t)
        pltpu.sync_copy(duf_hbm, du_t)
        iota = jnp.arange(LANES, dtype=jnp.int32)

        def group(g, _):
            sl = pl.ds(g * LANES, LANES)
            node = g * LANES + iota
            dvb = node * (3 * NBR)
            dub = node * NBR

            def loads(n):
                ix = dvb + 3 * n
                return (plsc.load_gather(dv_t, [ix]),
                        plsc.load_gather(dv_t, [ix + 1]),
                        plsc.load_gather(dv_t, [ix + 2]),
                        plsc.load_gather(du_t, [dub + n]))

            def store(gx, gy, gz):
                ox_t[sl] = gx
                oy_t[sl] = gy
                oz_t[sl] = gz

            _accum_solve(loads, store, g)
            return ()

        lax.fori_loop(0, TAIL // LANES, group, ())
        base = FULL_CHUNKS * CHUNK
        pltpu.sync_copy(ox_t, ox_hbm.at[pl.ds(base, TAIL)])
        pltpu.sync_copy(oy_t, oy_hbm.at[pl.ds(base, TAIL)])
        pltpu.sync_copy(oz_t, oz_hbm.at[pl.ds(base, TAIL)])

    nfull = FULL_CHUNKS // NUM_WORKERS

    def has_chunk(m):
        return wid + m * NUM_WORKERS < FULL_CHUNKS

    def round_one(m, slot, first):
        wait_in(slot)
        if not first:
            wait_out(slot)
        compute(slot)
        start_out(m, slot)
        nxt = m + 2

        @pl.when(has_chunk(nxt))
        def _refill():
            start_in(nxt, slot)

    start_in(0, 0)
    start_in(1, 1)
    round_one(0, 0, first=True)
    round_one(1, 1, first=True)

    def pair(k2, _):
        m = k2 * 2
        round_one(m, 0, first=False)
        round_one(m + 1, 1, first=False)
        return ()

    lax.fori_loop(1, nfull // 2, pair, ())

    wait_out(0)

    @pl.when(has_chunk(nfull))
    def _last():
        wait_in(0)
        compute(0)
        start_out(nfull, 0)
        wait_out(0)

    @pl.when(wid + nfull * NUM_WORKERS == FULL_CHUNKS)
    def _tail():
        process_tail()

    wait_out(1)


@jax.jit
def _ls_grads(dvt, dut, dvf, duf):
    f32 = jnp.float32
    run = pl.kernel(
        _body,
        out_type=(
            jax.ShapeDtypeStruct((N_NODES,), f32),
            jax.ShapeDtypeStruct((N_NODES,), f32),
            jax.ShapeDtypeStruct((N_NODES,), f32),
        ),
        mesh=plsc.VectorSubcoreMesh(core_axis_name="c", subcore_axis_name="s"),
        compiler_params=pltpu.CompilerParams(needs_layout_passes=False),
        scratch_types=[
            pltpu.VMEM((2, DIM, NBR, CHUNK), f32),
            pltpu.VMEM((2, NBR, CHUNK), f32),
            pltpu.VMEM((2, CHUNK), f32),
            pltpu.VMEM((2, CHUNK), f32),
            pltpu.VMEM((2, CHUNK), f32),
            pltpu.VMEM((TAIL * 3 * NBR,), f32),
            pltpu.VMEM((TAIL * NBR,), f32),
            pltpu.VMEM((TAIL,), f32),
            pltpu.VMEM((TAIL,), f32),
            pltpu.VMEM((TAIL,), f32),
            pltpu.SemaphoreType.DMA,
            pltpu.SemaphoreType.DMA,
            pltpu.SemaphoreType.DMA,
            pltpu.SemaphoreType.DMA,
        ],
    )
    return run(dvt, dut, dvf, duf)


def kernel(coords, connectivity_tensor, y, du, dv):
    del coords, connectivity_tensor, y
    dvt = jnp.transpose(dv, (2, 1, 0))
    dut = jnp.transpose(du[:, :, 0], (1, 0))
    base = FULL_CHUNKS * CHUNK
    dvf = dv[base:].reshape(-1)
    duf = du[base:].reshape(-1)
    gx, gy, gz = _ls_grads(dvt, dut, dvf, duf)
    return (gx[:, None], gy[:, None], gz[:, None])
